# Initial kernel scaffold; baseline (speedup 1.0000x reference)
import jax
import jax.numpy as jnp
from jax.experimental import pallas as pl


def kernel(x, ei_r0, ei_r1, ei_r2, W1_r0, b1_r0, W2_r0, b2_r0, W1_r1, b1_r1, W2_r1, b2_r1, W1_r2, b1_r2, W2_r2, b2_r2):
    raise NotImplementedError("write your pallas kernel here")


# speedup vs baseline: 1.0004x; 1.0004x over previous
import jax
import jax.numpy as jnp
from jax.experimental import pallas as pl

_BLK = 1000


def _mm3_kernel(x_ref, w0_ref, w1_ref, w2_ref, y0_ref, y1_ref, y2_ref):
    xb = x_ref[...]
    y0_ref[...] = jnp.dot(xb, w0_ref[...], preferred_element_type=jnp.float32)
    y1_ref[...] = jnp.dot(xb, w1_ref[...], preferred_element_type=jnp.float32)
    y2_ref[...] = jnp.dot(xb, w2_ref[...], preferred_element_type=jnp.float32)


def _relu_mm3_kernel(x_ref, w0_ref, w1_ref, w2_ref, y0_ref, y1_ref, y2_ref):
    xb = jnp.maximum(x_ref[...], 0.0)
    y0_ref[...] = jnp.dot(xb, w0_ref[...], preferred_element_type=jnp.float32)
    y1_ref[...] = jnp.dot(xb, w1_ref[...], preferred_element_type=jnp.float32)
    y2_ref[...] = jnp.dot(xb, w2_ref[...], preferred_element_type=jnp.float32)


def _mm3(x, w0, w1, w2, relu_in):
    n, d = x.shape
    dout = w0.shape[1]
    blk = _BLK if n % _BLK == 0 else n
    xspec = pl.BlockSpec((blk, d), lambda i: (i, 0))
    yspec = pl.BlockSpec((blk, dout), lambda i: (i, 0))
    wspec = pl.BlockSpec((d, dout), lambda i: (0, 0))
    return pl.pallas_call(
        _relu_mm3_kernel if relu_in else _mm3_kernel,
        grid=(n // blk,),
        in_specs=[xspec, wspec, wspec, wspec],
        out_specs=[yspec, yspec, yspec],
        out_shape=[jax.ShapeDtypeStruct((n, dout), jnp.float32)] * 3,
    )(x, w0, w1, w2)


def kernel(x, ei_r0, ei_r1, ei_r2,
           W1_r0, b1_r0, W2_r0, b2_r0,
           W1_r1, b1_r1, W2_r1, b2_r1,
           W1_r2, b1_r2, W2_r2, b2_r2):
    n = x.shape[0]
    eis = (ei_r0, ei_r1, ei_r2)
    rows = [ei[0] for ei in eis]
    cols = [ei[1] for ei in eis]
    norms = []
    for r in range(3):
        deg = jnp.zeros((n,), jnp.float32).at[cols[r]].add(1.0)
        dis = jnp.where(deg > 0, jax.lax.rsqrt(jnp.maximum(deg, 1e-12)), 0.0)
        norms.append(dis[rows[r]] * dis[cols[r]])

    y = _mm3(x, W1_r0, W1_r1, W1_r2, relu_in=False)
    pre = jnp.zeros((n, W1_r0.shape[1]), jnp.float32)
    for r in range(3):
        msg = y[r][rows[r]] * norms[r][:, None]
        pre = pre.at[cols[r]].add(msg)
    pre = pre + (b1_r0 + b1_r1 + b1_r2)

    z = _mm3(pre, W2_r0, W2_r1, W2_r2, relu_in=True)
    out = jnp.zeros((n, W2_r0.shape[1]), jnp.float32)
    for r in range(3):
        msg = z[r][rows[r]] * norms[r][:, None]
        out = out.at[cols[r]].add(msg)
    return out + (b2_r0 + b2_r1 + b2_r2)
